# baseline (speedup 1.0000x reference)
import jax
import jax.numpy as jnp
import numpy as np
from jax import lax
from jax.experimental import pallas as pl
from jax.experimental.pallas import tpu as pltpu
from jax.experimental.pallas import tpu_sc as plsc

OUT_DIM = 65536
NCROPS = 10
GLOBAL_CROPS = 2
WARMUP_TT = 0.04
TT = 0.07
WARMUP_EP = 30
NEPOCHS = 100
STUDENT_TEMP = 0.1
TOPK = 8
BATCH_PER_CROP = 64

N_STUDENT_ROWS = NCROPS * BATCH_PER_CROP
N_TEACHER_ROWS = GLOBAL_CROPS * BATCH_PER_CROP

SC_CORES = 2
SC_SUBCORES = 16

LANES = 16
N_BANDS = N_TEACHER_ROWS // 8
HALF_CT = 256
CT_PER_CHUNK = 32
N_CHUNKS = HALF_CT // CT_PER_CHUNK
CHUNK_W = CT_PER_CHUNK * 1024
CCHUNK_W = CT_PER_CHUNK * 128
CAND_CAP = 128
CAND_PAD = CAND_CAP + 16
NEG_INF = float("-inf")
GATH_PAD = 96


def _sc_body(tflat_hbm, sflat_hbm, center_hbm, valsO, colsO, gathO,
             rbA, rbB, cbA, cbB, candv, candi, idxb, gathb, stgv, stgi,
             ptr_smem, t2_smem, sem, gsem):
    cidx = lax.axis_index("c")
    sidx = lax.axis_index("s")
    band = sidx
    half = cidx
    iota = lax.broadcasted_iota(jnp.int32, (LANES,), 0)

    def chunk_dma(q, rbuf, cbuf):
        tbase = (band * 512 + half * HALF_CT + q * CT_PER_CHUNK) * 1024
        cbase = (half * HALF_CT + q * CT_PER_CHUNK) * 128
        return (pltpu.async_copy(tflat_hbm.at[pl.ds(tbase, CHUNK_W)], rbuf, sem),
                pltpu.async_copy(center_hbm.at[0, pl.ds(cbase, CCHUNK_W)],
                                 cbuf, sem))

    d0 = chunk_dma(0, rbA, cbA)
    d0[0].wait()
    d0[1].wait()
    d1 = chunk_dma(1, rbB, cbB)

    def init_cand(g, _):
        candv[pl.ds(g * LANES, LANES)] = jnp.full((LANES,), NEG_INF)
        candi[pl.ds(g * LANES, LANES)] = jnp.zeros((LANES,), jnp.int32)
        return 0

    lax.fori_loop(0, (8 * CAND_PAD) // LANES, init_cand, 0)

    def collect(p, base_slot, ds_, ms_, cols):
        for j in range(8):
            pos = plsc.cumsum(ms_[j].astype(jnp.int32))
            dest = base_slot + jnp.minimum(p + pos - 1, CAND_PAD - 1)
            plsc.store_scatter(candv, [dest], ds_[j], mask=ms_[j])
            plsc.store_scatter(candi, [dest], cols[j], mask=ms_[j])
            p = jnp.minimum(p + jnp.max(pos), CAND_CAP)
        return p

    col0 = (half * HALF_CT) * 128

    def _eighth_of_cand(sub):
        cvs = [candv[pl.ds(sub * CAND_PAD + j * LANES, LANES)]
               for j in range(CAND_PAD // LANES)]
        t2 = NEG_INF
        for k in range(8):
            m = cvs[0]
            for j in range(1, len(cvs)):
                m = jnp.maximum(m, cvs[j])
            t2 = jnp.max(m)
            if k < 7:
                cvs = [jnp.where(cv == t2, NEG_INF, cv) for cv in cvs]
        return t2

    def row0_body(sub, _):
        def grp_a(ctl, accs):
            off = ctl * 1024 + sub * 128
            coff = ctl * 128
            new = []
            for j in range(8):
                v = rbA[pl.ds(off + j * LANES, LANES)]
                c = cbA[pl.ds(coff + j * LANES, LANES)]
                d = v - c
                rbA[pl.ds(off + j * LANES, LANES)] = d
                new.append(jnp.maximum(accs[j], d))
            return tuple(new)

        acc = lax.fori_loop(0, CT_PER_CHUNK, grp_a,
                            (jnp.full((LANES,), NEG_INF),) * 8)

        work = list(acc)
        thresh = NEG_INF
        for k in range(8):
            m = work[0]
            for j in range(1, 8):
                m = jnp.maximum(m, work[j])
            thresh = jnp.max(m)
            if k < 7:
                work = [jnp.where(w == thresh, NEG_INF, w) for w in work]

        base_slot = sub * CAND_PAD

        def scan0(ctl, p):
            off = ctl * 1024 + sub * 128
            ds_ = [rbA[pl.ds(off + j * LANES, LANES)] for j in range(8)]
            ms_ = [d >= thresh for d in ds_]
            anym = ms_[0]
            for j in range(1, 8):
                anym = anym | ms_[j]
            cols = [iota + (col0 + ctl * 128 + j * LANES) for j in range(8)]
            return lax.cond(jnp.any(anym),
                            lambda pp: collect(pp, base_slot, ds_, ms_, cols),
                            lambda pp: pp, p)

        ptr = lax.fori_loop(0, CT_PER_CHUNK, scan0, jnp.int32(0))

        ptr_smem[sub] = ptr
        t2_smem[sub] = _eighth_of_cand(sub)
        return 0

    lax.fori_loop(0, 8, row0_body, 0)

    for q in range(1, N_CHUNKS):
        rbuf, cbuf = (rbB, cbB) if q % 2 == 1 else (rbA, cbA)
        dr, dc = d1 if q == 1 else dnext
        dr.wait()
        dc.wait()
        if q < N_CHUNKS - 1:
            nbufs = (rbA, cbA) if (q + 1) % 2 == 0 else (rbB, cbB)
            dnext = chunk_dma(q + 1, nbufs[0], nbufs[1])

        t2v = [jnp.zeros((LANES,), jnp.float32) + t2_smem[sub]
               for sub in range(8)]

        def scan_pair(i, _, q=q, rbuf=rbuf, cbuf=cbuf, t2v=t2v):
            cbase = col0 + q * CT_PER_CHUNK * 128

            trig = None
            for u in range(2):
                ctl = i * 2 + u
                cvs = [cbuf[pl.ds(ctl * 128 + j * LANES, LANES)]
                       for j in range(8)]
                for sub in range(8):
                    off = ctl * 1024 + sub * 128
                    m = None
                    for j in range(8):
                        d = rbuf[pl.ds(off + j * LANES, LANES)] - cvs[j]
                        m = d if m is None else jnp.maximum(m, d)
                    h = m >= t2v[sub]
                    trig = h if trig is None else (trig | h)

            def cold(___):
                def scan_row(sub, __):
                    t2 = t2_smem[sub]
                    p0 = ptr_smem[sub]
                    for u in range(2):
                        ctl = i * 2 + u
                        off = ctl * 1024 + sub * 128
                        ds_ = [rbuf[pl.ds(off + j * LANES, LANES)]
                               - cbuf[pl.ds(ctl * 128 + j * LANES, LANES)]
                               for j in range(8)]
                        m = ds_[0]
                        for j in range(1, 8):
                            m = jnp.maximum(m, ds_[j])

                        def coll(pp, ds_=ds_, ctl=ctl):
                            ms_ = [d >= t2 for d in ds_]
                            cols = [iota + (cbase + ctl * 128 + j * LANES)
                                    for j in range(8)]
                            return collect(pp, sub * CAND_PAD, ds_, ms_, cols)

                        p0 = lax.cond(jnp.any(m >= t2), coll,
                                      lambda pp: pp, p0)
                    ptr_smem[sub] = p0
                    return 0

                return lax.fori_loop(0, 8, scan_row, 0)

            lax.cond(jnp.any(trig), cold, lambda ___: 0, 0)
            return 0

        lax.fori_loop(0, CT_PER_CHUNK // 2, scan_pair, 0)

        if q in (1, 3):
            def refine_row(sub, _):
                t2_smem[sub] = _eighth_of_cand(sub)
                return 0

            lax.fori_loop(0, 8, refine_row, 0)

    def final_body(sub, _):
        r = band * 8 + sub
        bb = lax.rem(r, BATCH_PER_CROP)
        base_slot = sub * CAND_PAD

        cv = [candv[pl.ds(base_slot + j * LANES, LANES)]
              for j in range(CAND_PAD // LANES)]
        ci = [candi[pl.ds(base_slot + j * LANES, LANES)]
              for j in range(CAND_PAD // LANES)]
        BIG = jnp.int32(2 ** 30)
        tv = jnp.full((LANES,), NEG_INF)
        ti = jnp.zeros((LANES,), jnp.int32)
        for k in range(TOPK):
            m = cv[0]
            for j in range(1, len(cv)):
                m = jnp.maximum(m, cv[j])
            mx = jnp.max(m)
            cand_i = [jnp.where(cv[j] == mx, ci[j], BIG) for j in range(len(cv))]
            mn = cand_i[0]
            for j in range(1, len(cv)):
                mn = jnp.minimum(mn, cand_i[j])
            bi = jnp.min(mn)
            tv = jnp.where(iota == k, mx, tv)
            ti = jnp.where(iota == k, bi, ti)
            cv = [jnp.where((cv[j] == mx) & (ci[j] == bi), NEG_INF, cv[j])
                  for j in range(len(cv))]

        idxb[pl.ds(80, LANES)] = jnp.zeros((LANES,), jnp.int32)
        ct_g = lax.shift_right_logical(ti, 7)
        lane_g = ti & 127
        for v in range(NCROPS):
            sr = v * BATCH_PER_CROP + bb
            sband = lax.shift_right_logical(sr, 3)
            ssub = sr & 7
            poff = (sband * 512 + ct_g) * 1024 + ssub * 128 + lane_g
            idxb[pl.ds(v * TOPK, LANES)] = poff
        pltpu.async_copy(sflat_hbm.at[idxb], gathb, gsem).wait()

        out_off = (r * 2 + half) * LANES
        stgv[...] = tv
        stgi[...] = ti
        pltpu.sync_copy(stgv, valsO.at[pl.ds(out_off, LANES)])
        pltpu.sync_copy(stgi, colsO.at[pl.ds(out_off, LANES)])
        pltpu.sync_copy(gathb, gathO.at[pl.ds((r * 2 + half) * GATH_PAD,
                                              GATH_PAD)])
        return 0

    lax.fori_loop(0, 8, final_body, 0)


def _sc_sparse_stage(teacher, student, center):
    tflat = teacher.reshape(N_TEACHER_ROWS // 8, 8, OUT_DIM // 128, 128)
    tflat = tflat.transpose(0, 2, 1, 3).reshape(-1)
    sflat = student.reshape(N_STUDENT_ROWS // 8, 8, OUT_DIM // 128, 128)
    sflat = sflat.transpose(0, 2, 1, 3).reshape(-1)

    mesh = plsc.VectorSubcoreMesh(core_axis_name="c", subcore_axis_name="s",
                                  num_cores=SC_CORES, num_subcores=SC_SUBCORES)
    f = pl.kernel(
        _sc_body,
        out_type=[
            jax.ShapeDtypeStruct((N_TEACHER_ROWS * 2 * LANES,), jnp.float32),
            jax.ShapeDtypeStruct((N_TEACHER_ROWS * 2 * LANES,), jnp.int32),
            jax.ShapeDtypeStruct((N_TEACHER_ROWS * 2 * GATH_PAD,), jnp.float32),
        ],
        mesh=mesh,
        scratch_types=[
            pltpu.VMEM((CHUNK_W,), jnp.float32),
            pltpu.VMEM((CHUNK_W,), jnp.float32),
            pltpu.VMEM((CCHUNK_W,), jnp.float32),
            pltpu.VMEM((CCHUNK_W,), jnp.float32),
            pltpu.VMEM((8 * CAND_PAD,), jnp.float32),
            pltpu.VMEM((8 * CAND_PAD,), jnp.int32),
            pltpu.VMEM((GATH_PAD,), jnp.int32),
            pltpu.VMEM((GATH_PAD,), jnp.float32),
            pltpu.VMEM((LANES,), jnp.float32),
            pltpu.VMEM((LANES,), jnp.int32),
            pltpu.SMEM((16,), jnp.int32),
            pltpu.SMEM((16,), jnp.float32),
            pltpu.SemaphoreType.DMA,
            pltpu.SemaphoreType.DMA,
        ],
        compiler_params=pltpu.CompilerParams(needs_layout_passes=False),
    )
    return f(tflat, sflat, center)


ROW_BLK = 128
COL_BLK = 2048
N_COL_TILES = OUT_DIM // COL_BLK


def _lse_body(x_ref, out_ref, m_ref, s_ref):
    j = pl.program_id(1)

    @pl.when(j == 0)
    def _():
        m_ref[...] = jnp.full((ROW_BLK, 1), NEG_INF, jnp.float32)
        s_ref[...] = jnp.zeros((ROW_BLK, 1), jnp.float32)

    t = x_ref[...] * (1.0 / STUDENT_TEMP)
    tm = jnp.max(t, axis=1, keepdims=True)
    m_old = m_ref[...]
    m_new = jnp.maximum(m_old, tm)
    s_ref[...] = (s_ref[...] * jnp.exp(m_old - m_new)
                  + jnp.sum(jnp.exp(t - m_new), axis=1, keepdims=True))
    m_ref[...] = m_new

    @pl.when(j == N_COL_TILES - 1)
    def _():
        out_ref[...] = m_ref[...] + jnp.log(s_ref[...])


def _student_lse(student):
    return pl.pallas_call(
        _lse_body,
        grid=(N_STUDENT_ROWS // ROW_BLK, N_COL_TILES),
        in_specs=[pl.BlockSpec((ROW_BLK, COL_BLK), lambda i, j: (i, j))],
        out_specs=pl.BlockSpec((ROW_BLK, 1), lambda i, j: (i, 0)),
        out_shape=jax.ShapeDtypeStruct((N_STUDENT_ROWS, 1), jnp.float32),
        scratch_shapes=[
            pltpu.VMEM((ROW_BLK, 1), jnp.float32),
            pltpu.VMEM((ROW_BLK, 1), jnp.float32),
        ],
    )(student)


def _colsum_body(x_ref, out_ref):
    out_ref[...] = jnp.sum(x_ref[...], axis=0, keepdims=True)


def _teacher_colsum(teacher):
    return pl.pallas_call(
        _colsum_body,
        grid=(N_COL_TILES,),
        in_specs=[pl.BlockSpec((N_TEACHER_ROWS, COL_BLK), lambda j: (0, j))],
        out_specs=pl.BlockSpec((1, COL_BLK), lambda j: (0, j)),
        out_shape=jax.ShapeDtypeStruct((1, OUT_DIM), jnp.float32),
    )(teacher)


def _epilogue_body(lse_ref, vals_ref, cols_ref, gath_ref, colsum_ref,
                   center_ref, temp_ref, loss_ref, ent_ref, tent_ref):
    lse = lse_ref[...]
    valsO = vals_ref[...]
    colsO = cols_ref[...]
    gath = gath_ref[...]
    temp = temp_ref[...]

    vals16 = jnp.concatenate([valsO[:, 0:TOPK], valsO[:, 16:16 + TOPK]], axis=1)
    cols16 = jnp.concatenate([colsO[:, 0:TOPK], colsO[:, 16:16 + TOPK]], axis=1)
    BIGC = jnp.int32(2 ** 30)
    sel = jnp.zeros(vals16.shape, jnp.bool_)
    cur = vals16
    for _ in range(TOPK):
        mx = jnp.max(cur, axis=1, keepdims=True)
        is_mx = cur == mx
        mc = jnp.min(jnp.where(is_mx, cols16, BIGC), axis=1, keepdims=True)
        pick = is_mx & (cols16 == mc)
        sel = sel | pick
        cur = jnp.where(pick, NEG_INF, cur)

    mxv = jnp.max(jnp.where(sel, vals16, NEG_INF), axis=1, keepdims=True)
    e = jnp.where(sel, jnp.exp((vals16 - mxv) / temp), 0.0)
    p = e / jnp.sum(e, axis=1, keepdims=True)

    blocks = []
    zeros16 = jnp.zeros((N_TEACHER_ROWS, GATH_PAD - NCROPS * TOPK), jnp.float32)
    for h in range(2):
        ph = p[:, h * TOPK:(h + 1) * TOPK]
        blocks.append(jnp.concatenate([ph] * NCROPS + [zeros16], axis=1))
    w = jnp.concatenate(blocks, axis=1)

    col = lax.broadcasted_iota(jnp.int32, w.shape, 1)
    row = lax.broadcasted_iota(jnp.int32, w.shape, 0)
    vcol = lax.rem(col, GATH_PAD) // TOPK
    keep = ((vcol < NCROPS)
            & ~((row < BATCH_PER_CROP) & (vcol == 0))
            & ~((row >= BATCH_PER_CROP) & (vcol == 1)))
    g_total = jnp.sum(jnp.where(keep, w * gath, 0.0))

    rowi = lax.broadcasted_iota(jnp.int32, (N_STUDENT_ROWS, 1), 0)
    wl = jnp.where(rowi < GLOBAL_CROPS * BATCH_PER_CROP, 1.0, 2.0)
    lse_total = jnp.sum(wl * lse)

    n_terms = GLOBAL_CROPS * (NCROPS - 1)
    denom = n_terms * BATCH_PER_CROP
    loss_ref[...] = ((lse_total - g_total / STUDENT_TEMP) / denom).reshape(1, 1)

    c = center_ref[...]
    mcn = jnp.max(c)
    ec = jnp.exp(c - mcn)
    zc = jnp.sum(ec)
    lsm_c = c - (jnp.log(zc) + mcn)
    sm_c = ec / zc
    tent_ref[...] = jnp.sum(sm_c * lsm_c).reshape(1, 1)

    bc = colsum_ref[...] * (1.0 / N_TEACHER_ROWS)
    mb = jnp.max(bc)
    eb = jnp.exp(bc - mb)
    sm_b = eb / jnp.sum(eb)
    ent_ref[...] = jnp.sum(sm_b * lsm_c).reshape(1, 1)


def _epilogue(lse, vals, cols, gath, colsum, center, tempv):
    return pl.pallas_call(
        _epilogue_body,
        in_specs=[
            pl.BlockSpec((N_STUDENT_ROWS, 1), lambda: (0, 0)),
            pl.BlockSpec((N_TEACHER_ROWS, 32), lambda: (0, 0)),
            pl.BlockSpec((N_TEACHER_ROWS, 32), lambda: (0, 0)),
            pl.BlockSpec((N_TEACHER_ROWS, 2 * GATH_PAD), lambda: (0, 0)),
            pl.BlockSpec((1, OUT_DIM), lambda: (0, 0)),
            pl.BlockSpec((1, OUT_DIM), lambda: (0, 0)),
            pl.BlockSpec((1, 1), lambda: (0, 0)),
        ],
        out_specs=[
            pl.BlockSpec((1, 1), lambda: (0, 0)),
            pl.BlockSpec((1, 1), lambda: (0, 0)),
            pl.BlockSpec((1, 1), lambda: (0, 0)),
        ],
        out_shape=[
            jax.ShapeDtypeStruct((1, 1), jnp.float32),
            jax.ShapeDtypeStruct((1, 1), jnp.float32),
            jax.ShapeDtypeStruct((1, 1), jnp.float32),
        ],
    )(lse, vals, cols, gath, colsum, center, tempv)


def _teacher_temp_value(epoch):
    sched = np.concatenate((np.linspace(WARMUP_TT, TT, WARMUP_EP),
                            np.ones(NEPOCHS - WARMUP_EP) * TT))
    return jnp.asarray(sched, dtype=jnp.float32)[epoch]


def kernel(student_output, teacher_output, epoch, center):
    temp = _teacher_temp_value(epoch)
    tempv = temp.reshape(1, 1).astype(jnp.float32)

    vals, cols, gath = _sc_sparse_stage(teacher_output, student_output, center)
    vals = vals.reshape(N_TEACHER_ROWS, 2 * LANES)
    cols = cols.reshape(N_TEACHER_ROWS, 2 * LANES)
    gath = gath.reshape(N_TEACHER_ROWS, 2 * GATH_PAD)

    lse = _student_lse(student_output)
    colsum = _teacher_colsum(teacher_output)

    loss, ent, tent = _epilogue(lse, vals, cols, gath, colsum, center, tempv)
    return (loss.reshape(()), ent.reshape((1,)), tent.reshape((1,)))


# speedup vs baseline: 1.2880x; 1.2880x over previous
import jax
import jax.numpy as jnp
import numpy as np
from jax import lax
from jax.experimental import pallas as pl
from jax.experimental.pallas import tpu as pltpu
from jax.experimental.pallas import tpu_sc as plsc

OUT_DIM = 65536
NCROPS = 10
GLOBAL_CROPS = 2
WARMUP_TT = 0.04
TT = 0.07
WARMUP_EP = 30
NEPOCHS = 100
STUDENT_TEMP = 0.1
TOPK = 8
BATCH_PER_CROP = 64

N_STUDENT_ROWS = NCROPS * BATCH_PER_CROP
N_TEACHER_ROWS = GLOBAL_CROPS * BATCH_PER_CROP

SC_CORES = 2
SC_SUBCORES = 16

LANES = 16
N_BANDS = N_TEACHER_ROWS // 8
HALF_CT = 256
CT_PER_CHUNK = 32
N_CHUNKS = HALF_CT // CT_PER_CHUNK
CHUNK_W = CT_PER_CHUNK * 1024
CCHUNK_W = CT_PER_CHUNK * 128
CAND_CAP = 128
CAND_PAD = CAND_CAP + 16
NEG_INF = float("-inf")
GATH_PAD = 96


def _sc_body(tflat_hbm, sflat_hbm, center_hbm, valsO, colsO, gathO,
             rbA, rbB, cbA, cbB, candv, candi, idxb, gathb, stgv, stgi,
             ptr_smem, t2_smem, sem, gsem):
    cidx = lax.axis_index("c")
    sidx = lax.axis_index("s")
    band = sidx
    half = cidx
    iota = lax.broadcasted_iota(jnp.int32, (LANES,), 0)

    def chunk_dma(q, rbuf, cbuf):
        tbase = (band * 512 + half * HALF_CT + q * CT_PER_CHUNK) * 1024
        cbase = (half * HALF_CT + q * CT_PER_CHUNK) * 128
        return (pltpu.async_copy(tflat_hbm.at[pl.ds(tbase, CHUNK_W)], rbuf, sem),
                pltpu.async_copy(center_hbm.at[0, pl.ds(cbase, CCHUNK_W)],
                                 cbuf, sem))

    d0 = chunk_dma(0, rbA, cbA)
    d0[0].wait()
    d0[1].wait()
    d1 = chunk_dma(1, rbB, cbB)

    def init_cand(g, _):
        candv[pl.ds(g * LANES, LANES)] = jnp.full((LANES,), NEG_INF)
        candi[pl.ds(g * LANES, LANES)] = jnp.zeros((LANES,), jnp.int32)
        return 0

    lax.fori_loop(0, (8 * CAND_PAD) // LANES, init_cand, 0)

    def collect(p, base_slot, ds_, ms_, cols):
        for j in range(8):
            pos = plsc.cumsum(ms_[j].astype(jnp.int32))
            dest = base_slot + jnp.minimum(p + pos - 1, CAND_PAD - 1)
            plsc.store_scatter(candv, [dest], ds_[j], mask=ms_[j])
            plsc.store_scatter(candi, [dest], cols[j], mask=ms_[j])
            p = jnp.minimum(p + jnp.max(pos), CAND_CAP)
        return p

    col0 = (half * HALF_CT) * 128

    def _eighth_of_cand(sub):
        cvs = [candv[pl.ds(sub * CAND_PAD + j * LANES, LANES)]
               for j in range(CAND_PAD // LANES)]
        t2 = NEG_INF
        for k in range(8):
            m = cvs[0]
            for j in range(1, len(cvs)):
                m = jnp.maximum(m, cvs[j])
            t2 = jnp.max(m)
            if k < 7:
                cvs = [jnp.where(cv == t2, NEG_INF, cv) for cv in cvs]
        return t2

    def row0_body(sub, _):
        def grp_a(ctl, accs):
            off = ctl * 1024 + sub * 128
            coff = ctl * 128
            new = []
            for j in range(8):
                v = rbA[pl.ds(off + j * LANES, LANES)]
                c = cbA[pl.ds(coff + j * LANES, LANES)]
                d = v - c
                rbA[pl.ds(off + j * LANES, LANES)] = d
                new.append(jnp.maximum(accs[j], d))
            return tuple(new)

        acc = lax.fori_loop(0, CT_PER_CHUNK, grp_a,
                            (jnp.full((LANES,), NEG_INF),) * 8)

        work = list(acc)
        thresh = NEG_INF
        for k in range(8):
            m = work[0]
            for j in range(1, 8):
                m = jnp.maximum(m, work[j])
            thresh = jnp.max(m)
            if k < 7:
                work = [jnp.where(w == thresh, NEG_INF, w) for w in work]

        base_slot = sub * CAND_PAD

        def scan0(ctl, p):
            off = ctl * 1024 + sub * 128
            ds_ = [rbA[pl.ds(off + j * LANES, LANES)] for j in range(8)]
            ms_ = [d >= thresh for d in ds_]
            anym = ms_[0]
            for j in range(1, 8):
                anym = anym | ms_[j]
            cols = [iota + (col0 + ctl * 128 + j * LANES) for j in range(8)]
            return lax.cond(jnp.any(anym),
                            lambda pp: collect(pp, base_slot, ds_, ms_, cols),
                            lambda pp: pp, p)

        ptr = lax.fori_loop(0, CT_PER_CHUNK, scan0, jnp.int32(0))

        ptr_smem[sub] = ptr
        t2_smem[sub] = _eighth_of_cand(sub)
        return 0

    lax.fori_loop(0, 8, row0_body, 0)

    for q in range(1, N_CHUNKS):
        rbuf, cbuf = (rbB, cbB) if q % 2 == 1 else (rbA, cbA)
        dr, dc = d1 if q == 1 else dnext
        dr.wait()
        dc.wait()
        if q < N_CHUNKS - 1:
            nbufs = (rbA, cbA) if (q + 1) % 2 == 0 else (rbB, cbB)
            dnext = chunk_dma(q + 1, nbufs[0], nbufs[1])

        t2v = [jnp.zeros((LANES,), jnp.float32) + t2_smem[sub]
               for sub in range(8)]

        def scan_pair(i, _, q=q, rbuf=rbuf, cbuf=cbuf, t2v=t2v):
            cbase = col0 + q * CT_PER_CHUNK * 128

            trig = None
            for u in range(2):
                ctl = i * 2 + u
                cvs = [cbuf[pl.ds(ctl * 128 + j * LANES, LANES)]
                       for j in range(8)]
                for sub in range(8):
                    off = ctl * 1024 + sub * 128
                    m = None
                    for j in range(8):
                        d = rbuf[pl.ds(off + j * LANES, LANES)] - cvs[j]
                        m = d if m is None else jnp.maximum(m, d)
                    h = m >= t2v[sub]
                    trig = h if trig is None else (trig | h)

            def cold(___):
                def scan_row(sub, __):
                    t2 = t2_smem[sub]
                    p0 = ptr_smem[sub]
                    for u in range(2):
                        ctl = i * 2 + u
                        off = ctl * 1024 + sub * 128
                        ds_ = [rbuf[pl.ds(off + j * LANES, LANES)]
                               - cbuf[pl.ds(ctl * 128 + j * LANES, LANES)]
                               for j in range(8)]
                        m = ds_[0]
                        for j in range(1, 8):
                            m = jnp.maximum(m, ds_[j])

                        def coll(pp, ds_=ds_, ctl=ctl):
                            ms_ = [d >= t2 for d in ds_]
                            cols = [iota + (cbase + ctl * 128 + j * LANES)
                                    for j in range(8)]
                            return collect(pp, sub * CAND_PAD, ds_, ms_, cols)

                        p0 = lax.cond(jnp.any(m >= t2), coll,
                                      lambda pp: pp, p0)
                    ptr_smem[sub] = p0
                    return 0

                return lax.fori_loop(0, 8, scan_row, 0)

            lax.cond(jnp.any(trig), cold, lambda ___: 0, 0)
            return 0

        lax.fori_loop(0, CT_PER_CHUNK // 2, scan_pair, 0)

        if q in (1, 3):
            def refine_row(sub, _):
                t2_smem[sub] = _eighth_of_cand(sub)
                return 0

            lax.fori_loop(0, 8, refine_row, 0)

    def final_body(sub, _):
        r = band * 8 + sub
        bb = lax.rem(r, BATCH_PER_CROP)
        base_slot = sub * CAND_PAD

        cv = [candv[pl.ds(base_slot + j * LANES, LANES)]
              for j in range(CAND_PAD // LANES)]
        ci = [candi[pl.ds(base_slot + j * LANES, LANES)]
              for j in range(CAND_PAD // LANES)]
        BIG = jnp.int32(2 ** 30)
        tv = jnp.full((LANES,), NEG_INF)
        ti = jnp.zeros((LANES,), jnp.int32)
        for k in range(TOPK):
            m = cv[0]
            for j in range(1, len(cv)):
                m = jnp.maximum(m, cv[j])
            mx = jnp.max(m)
            cand_i = [jnp.where(cv[j] == mx, ci[j], BIG) for j in range(len(cv))]
            mn = cand_i[0]
            for j in range(1, len(cv)):
                mn = jnp.minimum(mn, cand_i[j])
            bi = jnp.min(mn)
            tv = jnp.where(iota == k, mx, tv)
            ti = jnp.where(iota == k, bi, ti)
            cv = [jnp.where((cv[j] == mx) & (ci[j] == bi), NEG_INF, cv[j])
                  for j in range(len(cv))]

        idxb[pl.ds(80, LANES)] = jnp.zeros((LANES,), jnp.int32)
        ct_g = lax.shift_right_logical(ti, 7)
        lane_g = ti & 127
        for v in range(NCROPS):
            sr = v * BATCH_PER_CROP + bb
            sband = lax.shift_right_logical(sr, 3)
            ssub = sr & 7
            poff = (sband * 512 + ct_g) * 1024 + ssub * 128 + lane_g
            idxb[pl.ds(v * TOPK, LANES)] = poff
        pltpu.async_copy(sflat_hbm.at[idxb], gathb, gsem).wait()

        out_off = (r * 2 + half) * LANES
        stgv[...] = tv
        stgi[...] = ti
        pltpu.sync_copy(stgv, valsO.at[pl.ds(out_off, LANES)])
        pltpu.sync_copy(stgi, colsO.at[pl.ds(out_off, LANES)])
        pltpu.sync_copy(gathb, gathO.at[pl.ds((r * 2 + half) * GATH_PAD,
                                              GATH_PAD)])
        return 0

    lax.fori_loop(0, 8, final_body, 0)


def _sc_sparse_stage(teacher, student, center):
    tflat = teacher.reshape(N_TEACHER_ROWS // 8, 8, OUT_DIM // 128, 128)
    tflat = tflat.transpose(0, 2, 1, 3).reshape(-1)
    sflat = student.reshape(N_STUDENT_ROWS // 8, 8, OUT_DIM // 128, 128)
    sflat = sflat.transpose(0, 2, 1, 3).reshape(-1)

    mesh = plsc.VectorSubcoreMesh(core_axis_name="c", subcore_axis_name="s",
                                  num_cores=SC_CORES, num_subcores=SC_SUBCORES)
    f = pl.kernel(
        _sc_body,
        out_type=[
            jax.ShapeDtypeStruct((N_TEACHER_ROWS * 2 * LANES,), jnp.float32),
            jax.ShapeDtypeStruct((N_TEACHER_ROWS * 2 * LANES,), jnp.int32),
            jax.ShapeDtypeStruct((N_TEACHER_ROWS * 2 * GATH_PAD,), jnp.float32),
        ],
        mesh=mesh,
        scratch_types=[
            pltpu.VMEM((CHUNK_W,), jnp.float32),
            pltpu.VMEM((CHUNK_W,), jnp.float32),
            pltpu.VMEM((CCHUNK_W,), jnp.float32),
            pltpu.VMEM((CCHUNK_W,), jnp.float32),
            pltpu.VMEM((8 * CAND_PAD,), jnp.float32),
            pltpu.VMEM((8 * CAND_PAD,), jnp.int32),
            pltpu.VMEM((GATH_PAD,), jnp.int32),
            pltpu.VMEM((GATH_PAD,), jnp.float32),
            pltpu.VMEM((LANES,), jnp.float32),
            pltpu.VMEM((LANES,), jnp.int32),
            pltpu.SMEM((16,), jnp.int32),
            pltpu.SMEM((16,), jnp.float32),
            pltpu.SemaphoreType.DMA,
            pltpu.SemaphoreType.DMA,
        ],
        compiler_params=pltpu.CompilerParams(needs_layout_passes=False),
    )
    return f(tflat, sflat, center)


ROW_BLK = 128
COL_BLK = 4096
N_COL_TILES = OUT_DIM // COL_BLK


def _lse_body(x_ref, out_ref, m_ref, s_ref):
    j = pl.program_id(1)

    @pl.when(j == 0)
    def _():
        m_ref[...] = jnp.full((ROW_BLK, 1), NEG_INF, jnp.float32)
        s_ref[...] = jnp.zeros((ROW_BLK, 1), jnp.float32)

    t = x_ref[...] * (1.0 / STUDENT_TEMP)
    tm = jnp.max(t, axis=1, keepdims=True)
    m_old = m_ref[...]
    m_new = jnp.maximum(m_old, tm)
    s_ref[...] = (s_ref[...] * jnp.exp(m_old - m_new)
                  + jnp.sum(jnp.exp(t - m_new), axis=1, keepdims=True))
    m_ref[...] = m_new

    @pl.when(j == N_COL_TILES - 1)
    def _():
        out_ref[...] = m_ref[...] + jnp.log(s_ref[...])


def _student_lse(student):
    return pl.pallas_call(
        _lse_body,
        grid=(N_STUDENT_ROWS // ROW_BLK, N_COL_TILES),
        in_specs=[pl.BlockSpec((ROW_BLK, COL_BLK), lambda i, j: (i, j))],
        out_specs=pl.BlockSpec((ROW_BLK, 1), lambda i, j: (i, 0)),
        out_shape=jax.ShapeDtypeStruct((N_STUDENT_ROWS, 1), jnp.float32),
        scratch_shapes=[
            pltpu.VMEM((ROW_BLK, 1), jnp.float32),
            pltpu.VMEM((ROW_BLK, 1), jnp.float32),
        ],
        compiler_params=pltpu.CompilerParams(
            dimension_semantics=("parallel", "arbitrary")),
    )(student)


def _colsum_body(x_ref, out_ref):
    out_ref[...] = jnp.sum(x_ref[...], axis=0, keepdims=True)


def _teacher_colsum(teacher):
    return pl.pallas_call(
        _colsum_body,
        grid=(N_COL_TILES,),
        in_specs=[pl.BlockSpec((N_TEACHER_ROWS, COL_BLK), lambda j: (0, j))],
        out_specs=pl.BlockSpec((1, COL_BLK), lambda j: (0, j)),
        out_shape=jax.ShapeDtypeStruct((1, OUT_DIM), jnp.float32),
    )(teacher)


def _epilogue_body(lse_ref, vals_ref, cols_ref, gath_ref, colsum_ref,
                   center_ref, temp_ref, loss_ref, ent_ref, tent_ref):
    lse = lse_ref[...]
    valsO = vals_ref[...]
    colsO = cols_ref[...]
    gath = gath_ref[...]
    temp = temp_ref[...]

    vals16 = jnp.concatenate([valsO[:, 0:TOPK], valsO[:, 16:16 + TOPK]], axis=1)
    cols16 = jnp.concatenate([colsO[:, 0:TOPK], colsO[:, 16:16 + TOPK]], axis=1)
    BIGC = jnp.int32(2 ** 30)
    sel = jnp.zeros(vals16.shape, jnp.bool_)
    cur = vals16
    for _ in range(TOPK):
        mx = jnp.max(cur, axis=1, keepdims=True)
        is_mx = cur == mx
        mc = jnp.min(jnp.where(is_mx, cols16, BIGC), axis=1, keepdims=True)
        pick = is_mx & (cols16 == mc)
        sel = sel | pick
        cur = jnp.where(pick, NEG_INF, cur)

    mxv = jnp.max(jnp.where(sel, vals16, NEG_INF), axis=1, keepdims=True)
    e = jnp.where(sel, jnp.exp((vals16 - mxv) / temp), 0.0)
    p = e / jnp.sum(e, axis=1, keepdims=True)

    blocks = []
    zeros16 = jnp.zeros((N_TEACHER_ROWS, GATH_PAD - NCROPS * TOPK), jnp.float32)
    for h in range(2):
        ph = p[:, h * TOPK:(h + 1) * TOPK]
        blocks.append(jnp.concatenate([ph] * NCROPS + [zeros16], axis=1))
    w = jnp.concatenate(blocks, axis=1)

    col = lax.broadcasted_iota(jnp.int32, w.shape, 1)
    row = lax.broadcasted_iota(jnp.int32, w.shape, 0)
    vcol = lax.rem(col, GATH_PAD) // TOPK
    keep = ((vcol < NCROPS)
            & ~((row < BATCH_PER_CROP) & (vcol == 0))
            & ~((row >= BATCH_PER_CROP) & (vcol == 1)))
    g_total = jnp.sum(jnp.where(keep, w * gath, 0.0))

    rowi = lax.broadcasted_iota(jnp.int32, (N_STUDENT_ROWS, 1), 0)
    wl = jnp.where(rowi < GLOBAL_CROPS * BATCH_PER_CROP, 1.0, 2.0)
    lse_total = jnp.sum(wl * lse)

    n_terms = GLOBAL_CROPS * (NCROPS - 1)
    denom = n_terms * BATCH_PER_CROP
    loss_ref[...] = ((lse_total - g_total / STUDENT_TEMP) / denom).reshape(1, 1)

    c = center_ref[...]
    mcn = jnp.max(c)
    ec = jnp.exp(c - mcn)
    zc = jnp.sum(ec)
    lsm_c = c - (jnp.log(zc) + mcn)
    sm_c = ec / zc
    tent_ref[...] = jnp.sum(sm_c * lsm_c).reshape(1, 1)

    bc = colsum_ref[...] * (1.0 / N_TEACHER_ROWS)
    mb = jnp.max(bc)
    eb = jnp.exp(bc - mb)
    sm_b = eb / jnp.sum(eb)
    ent_ref[...] = jnp.sum(sm_b * lsm_c).reshape(1, 1)


def _epilogue(lse, vals, cols, gath, colsum, center, tempv):
    return pl.pallas_call(
        _epilogue_body,
        in_specs=[
            pl.BlockSpec((N_STUDENT_ROWS, 1), lambda: (0, 0)),
            pl.BlockSpec((N_TEACHER_ROWS, 32), lambda: (0, 0)),
            pl.BlockSpec((N_TEACHER_ROWS, 32), lambda: (0, 0)),
            pl.BlockSpec((N_TEACHER_ROWS, 2 * GATH_PAD), lambda: (0, 0)),
            pl.BlockSpec((1, OUT_DIM), lambda: (0, 0)),
            pl.BlockSpec((1, OUT_DIM), lambda: (0, 0)),
            pl.BlockSpec((1, 1), lambda: (0, 0)),
        ],
        out_specs=[
            pl.BlockSpec((1, 1), lambda: (0, 0)),
            pl.BlockSpec((1, 1), lambda: (0, 0)),
            pl.BlockSpec((1, 1), lambda: (0, 0)),
        ],
        out_shape=[
            jax.ShapeDtypeStruct((1, 1), jnp.float32),
            jax.ShapeDtypeStruct((1, 1), jnp.float32),
            jax.ShapeDtypeStruct((1, 1), jnp.float32),
        ],
    )(lse, vals, cols, gath, colsum, center, tempv)


def _teacher_temp_value(epoch):
    sched = np.concatenate((np.linspace(WARMUP_TT, TT, WARMUP_EP),
                            np.ones(NEPOCHS - WARMUP_EP) * TT))
    return jnp.asarray(sched, dtype=jnp.float32)[epoch]


def kernel(student_output, teacher_output, epoch, center):
    temp = _teacher_temp_value(epoch)
    tempv = temp.reshape(1, 1).astype(jnp.float32)

    vals, cols, gath = _sc_sparse_stage(teacher_output, student_output, center)
    vals = vals.reshape(N_TEACHER_ROWS, 2 * LANES)
    cols = cols.reshape(N_TEACHER_ROWS, 2 * LANES)
    gath = gath.reshape(N_TEACHER_ROWS, 2 * GATH_PAD)

    lse = _student_lse(student_output)
    colsum = _teacher_colsum(teacher_output)

    loss, ent, tent = _epilogue(lse, vals, cols, gath, colsum, center, tempv)
    return (loss.reshape(()), ent.reshape((1,)), tent.reshape((1,)))
